# baseline (speedup 1.0000x reference)
import functools

import jax
import jax.numpy as jnp
from jax import lax
from jax.experimental import pallas as pl
from jax.experimental.pallas import tpu as pltpu
from jax.experimental.pallas import tpu_sc as plsc

_NUM_CORES = 2
_NUM_SUBCORES = 16
_NW = _NUM_CORES * _NUM_SUBCORES


@functools.partial(jax.jit, static_argnums=(1, 2))
def _sc_launder_idx(idx2d, N, L):
    r_per_w = N // _NW
    mesh = plsc.VectorSubcoreMesh(core_axis_name="c", subcore_axis_name="s")

    @functools.partial(
        pl.kernel,
        out_type=jax.ShapeDtypeStruct((N, L), jnp.int32),
        mesh=mesh,
        compiler_params=pltpu.CompilerParams(use_tc_tiling_on_sc=False),
        scratch_types=[pltpu.VMEM((r_per_w, L), jnp.int32)],
    )
    def launder_kernel(idx_hbm, out_hbm, idx_v):
        wid = lax.axis_index("s") * _NUM_CORES + lax.axis_index("c")
        rbase = wid * r_per_w
        pltpu.sync_copy(idx_hbm.at[pl.ds(rbase, r_per_w)], idx_v)
        pltpu.sync_copy(idx_v, out_hbm.at[pl.ds(rbase, r_per_w)])

    return launder_kernel(idx2d)


@functools.partial(jax.jit, static_argnums=(2, 3, 4))
def _sc_gather(table, idx, B, D, CH):
    b_per_w = B // _NW
    n_ch = b_per_w // CH
    mesh = plsc.VectorSubcoreMesh(core_axis_name="c", subcore_axis_name="s")

    @functools.partial(
        pl.kernel,
        out_type=jax.ShapeDtypeStruct((B, D), jnp.float32),
        mesh=mesh,
        compiler_params=pltpu.CompilerParams(use_tc_tiling_on_sc=False),
        scratch_types=[
            pltpu.VMEM((b_per_w,), jnp.int32),
            pltpu.VMEM((CH, D), jnp.float32),
            pltpu.SemaphoreType.DMA,
        ],
    )
    def gather_kernel(t_hbm, idx_hbm, out_hbm, idx_v, rows_v, sem):
        wid = lax.axis_index("s") * _NUM_CORES + lax.axis_index("c")
        base = wid * b_per_w
        pltpu.sync_copy(idx_hbm.at[pl.ds(base, b_per_w)], idx_v)

        def body(c, carry):
            off = c * CH
            pltpu.async_copy(
                t_hbm.at[idx_v.at[pl.ds(off, CH)]], rows_v, sem
            ).wait()
            pltpu.sync_copy(rows_v, out_hbm.at[pl.ds(base + off, CH)])
            return carry

        lax.fori_loop(0, n_ch, body, 0)

    return gather_kernel(table, idx)


def _adapter_body(g_ref, w_ref, b_ref, o_ref):
    rows = jnp.dot(g_ref[...], w_ref[...], preferred_element_type=jnp.float32)
    rows = rows + b_ref[...]
    o_ref[...] = rows.reshape(o_ref.shape)


def _adapter(g, W, b, N, L, D):
    BLK = 256
    assert N % BLK == 0
    return pl.pallas_call(
        _adapter_body,
        grid=(N // BLK,),
        in_specs=[
            pl.BlockSpec((BLK * L, D), lambda i: (i, 0)),
            pl.BlockSpec((D, D), lambda i: (0, 0)),
            pl.BlockSpec((1, D), lambda i: (0, 0)),
        ],
        out_specs=pl.BlockSpec((BLK, L, D), lambda i: (i, 0, 0)),
        out_shape=jax.ShapeDtypeStruct((N, L, D), jnp.float32),
    )(g, W, b.reshape(1, D))


def kernel(indices, table, W, b):
    V, D = table.shape
    N, L = indices.shape
    idx = _sc_launder_idx(indices.astype(jnp.int32), N, L).reshape(-1)
    g = _sc_gather(table, idx, N * L, D, 1280)
    return _adapter(g, W, b, N, L, D)


# speedup vs baseline: 1.0396x; 1.0396x over previous
import functools

import jax
import jax.numpy as jnp
from jax import lax
from jax.experimental import pallas as pl
from jax.experimental.pallas import tpu as pltpu
from jax.experimental.pallas import tpu_sc as plsc

_NUM_CORES = 2
_NUM_SUBCORES = 16
_NW = _NUM_CORES * _NUM_SUBCORES
_LANES = 16


@functools.partial(jax.jit, static_argnums=(2, 3, 4))
def _sc_gather_flat(table, idx, B, D, CH):
    b_per_w = B // _NW
    n_ch = b_per_w // CH
    mesh = plsc.VectorSubcoreMesh(core_axis_name="c", subcore_axis_name="s")

    @functools.partial(
        pl.kernel,
        out_type=jax.ShapeDtypeStruct((B, D), jnp.float32),
        mesh=mesh,
        compiler_params=pltpu.CompilerParams(use_tc_tiling_on_sc=False),
        scratch_types=[
            pltpu.VMEM((b_per_w,), jnp.int32),
            pltpu.VMEM((CH, D), jnp.float32),
            pltpu.SemaphoreType.DMA,
        ],
    )
    def gather_kernel(t_hbm, idx_hbm, out_hbm, idx_v, rows_v, sem):
        wid = lax.axis_index("s") * _NUM_CORES + lax.axis_index("c")
        base = wid * b_per_w
        pltpu.sync_copy(idx_hbm.at[pl.ds(base, b_per_w)], idx_v)

        def body(c, carry):
            off = c * CH
            pltpu.async_copy(
                t_hbm.at[idx_v.at[pl.ds(off, CH)]], rows_v, sem
            ).wait()
            pltpu.sync_copy(rows_v, out_hbm.at[pl.ds(base + off, CH)])
            return carry

        lax.fori_loop(0, n_ch, body, 0)

    return gather_kernel(table, idx)


@functools.partial(jax.jit, static_argnums=(2, 3, 4))
def _sc_gather(table, idx_p, N, L, D):
    LP = idx_p.shape[1]
    r_per_w = N // _NW
    HALF = r_per_w // 2
    n_half = HALF * L
    CH = 1600
    n_ch = n_half // CH
    n_cp = n_half // _LANES
    mesh = plsc.VectorSubcoreMesh(core_axis_name="c", subcore_axis_name="s")

    @functools.partial(
        pl.kernel,
        out_type=jax.ShapeDtypeStruct((N * L, D), jnp.float32),
        mesh=mesh,
        compiler_params=pltpu.CompilerParams(use_tc_tiling_on_sc=False),
        scratch_types=[
            pltpu.VMEM((HALF, LP), jnp.int32),
            pltpu.VMEM((n_half,), jnp.int32),
            pltpu.VMEM((CH, D), jnp.float32),
            pltpu.SemaphoreType.DMA,
        ],
    )
    def gather_kernel(t_hbm, idx_hbm, out_hbm, slab_v, idxc_v, rows_v, sem):
        wid = lax.axis_index("s") * _NUM_CORES + lax.axis_index("c")
        rbase = wid * r_per_w
        lane = lax.iota(jnp.int32, _LANES)

        for h in range(2):
            pltpu.sync_copy(idx_hbm.at[pl.ds(rbase + h * HALF, HALF)], slab_v)

            def compact(j, carry):
                p = j * _LANES + lane
                vals = plsc.load_gather(slab_v, [p // L, p % L])
                idxc_v[pl.ds(j * _LANES, _LANES)] = vals
                return carry

            lax.fori_loop(0, n_cp, compact, 0)

            def body(c, carry):
                off = c * CH
                pltpu.async_copy(
                    t_hbm.at[idxc_v.at[pl.ds(off, CH)]], rows_v, sem
                ).wait()
                pltpu.sync_copy(
                    rows_v,
                    out_hbm.at[pl.ds((rbase + h * HALF) * L + off, CH)],
                )
                return carry

            lax.fori_loop(0, n_ch, body, 0)

    return gather_kernel(table, idx_p)


def _adapter_body(g_ref, w_ref, b_ref, o_ref):
    g = g_ref[...]
    w = w_ref[...]
    nk = g.shape[1] // w.shape[0]
    pieces = [
        jnp.dot(
            g[:, k * w.shape[0] : (k + 1) * w.shape[0]],
            w,
            preferred_element_type=jnp.float32,
        )
        for k in range(nk)
    ]
    y = jnp.stack(pieces, axis=1) + b_ref[...]
    o_ref[...] = y.reshape(o_ref.shape)


def _adapter(g128, W128, b128, N, L, D):
    BLK = 256
    XB = BLK * L * D // 128
    assert N % BLK == 0
    return pl.pallas_call(
        _adapter_body,
        grid=(N // BLK,),
        in_specs=[
            pl.BlockSpec((XB, 128), lambda i: (i, 0)),
            pl.BlockSpec((D, D), lambda i: (0, 0)),
            pl.BlockSpec((1, D), lambda i: (0, 0)),
        ],
        out_specs=pl.BlockSpec((BLK, L, D), lambda i: (i, 0, 0)),
        out_shape=jax.ShapeDtypeStruct((N, L, D), jnp.float32),
    )(g128, W128, b128.reshape(1, D))


def kernel(indices, table, W, b):
    V, D = table.shape
    N, L = indices.shape
    idx = indices.reshape(-1).astype(jnp.int32)
    g = _sc_gather_flat(table, idx, N * L, D, 1600)
    g128 = g.reshape(N * L * D // 128, 128)
    return _adapter(g128, W, b, N, L, D)


# speedup vs baseline: 1.2816x; 1.2328x over previous
import functools

import jax
import jax.numpy as jnp
from jax import lax
from jax.experimental import pallas as pl
from jax.experimental.pallas import tpu as pltpu
from jax.experimental.pallas import tpu_sc as plsc

_NUM_CORES = 2
_NUM_SUBCORES = 16
_NW = _NUM_CORES * _NUM_SUBCORES
_LANES = 16


@functools.partial(jax.jit, static_argnums=(2, 3, 4))
def _sc_gather_flat(table, idx, B, D, CH):
    b_per_w = B // _NW
    n_ch = b_per_w // CH
    mesh = plsc.VectorSubcoreMesh(core_axis_name="c", subcore_axis_name="s")

    @functools.partial(
        pl.kernel,
        out_type=jax.ShapeDtypeStruct((B, D), jnp.float32),
        mesh=mesh,
        compiler_params=pltpu.CompilerParams(use_tc_tiling_on_sc=False),
        scratch_types=[
            pltpu.VMEM((b_per_w,), jnp.int32),
            pltpu.VMEM((CH, D), jnp.float32),
            pltpu.SemaphoreType.DMA,
        ],
    )
    def gather_kernel(t_hbm, idx_hbm, out_hbm, idx_v, rows_v, sem):
        wid = lax.axis_index("s") * _NUM_CORES + lax.axis_index("c")
        base = wid * b_per_w
        pltpu.sync_copy(idx_hbm.at[pl.ds(base, b_per_w)], idx_v)

        def body(c, carry):
            off = c * CH
            pltpu.async_copy(
                t_hbm.at[idx_v.at[pl.ds(off, CH)]], rows_v, sem
            ).wait()
            pltpu.sync_copy(rows_v, out_hbm.at[pl.ds(base + off, CH)])
            return carry

        lax.fori_loop(0, n_ch, body, 0)

    return gather_kernel(table, idx)


@functools.partial(jax.jit, static_argnums=(2, 3, 4))
def _sc_gather(table, idx_p, N, L, D):
    LP = idx_p.shape[1]
    r_per_w = N // _NW
    HALF = r_per_w // 2
    n_half = HALF * L
    CH = 1600
    n_ch = n_half // CH
    n_cp = n_half // _LANES
    mesh = plsc.VectorSubcoreMesh(core_axis_name="c", subcore_axis_name="s")

    @functools.partial(
        pl.kernel,
        out_type=jax.ShapeDtypeStruct((N * L, D), jnp.float32),
        mesh=mesh,
        compiler_params=pltpu.CompilerParams(use_tc_tiling_on_sc=False),
        scratch_types=[
            pltpu.VMEM((HALF, LP), jnp.int32),
            pltpu.VMEM((n_half,), jnp.int32),
            pltpu.VMEM((CH, D), jnp.float32),
            pltpu.SemaphoreType.DMA,
        ],
    )
    def gather_kernel(t_hbm, idx_hbm, out_hbm, slab_v, idxc_v, rows_v, sem):
        wid = lax.axis_index("s") * _NUM_CORES + lax.axis_index("c")
        rbase = wid * r_per_w
        lane = lax.iota(jnp.int32, _LANES)

        for h in range(2):
            pltpu.sync_copy(idx_hbm.at[pl.ds(rbase + h * HALF, HALF)], slab_v)

            def compact(j, carry):
                p = j * _LANES + lane
                vals = plsc.load_gather(slab_v, [p // L, p % L])
                idxc_v[pl.ds(j * _LANES, _LANES)] = vals
                return carry

            lax.fori_loop(0, n_cp, compact, 0)

            def body(c, carry):
                off = c * CH
                pltpu.async_copy(
                    t_hbm.at[idxc_v.at[pl.ds(off, CH)]], rows_v, sem
                ).wait()
                pltpu.sync_copy(
                    rows_v,
                    out_hbm.at[pl.ds((rbase + h * HALF) * L + off, CH)],
                )
                return carry

            lax.fori_loop(0, n_ch, body, 0)

    return gather_kernel(table, idx_p)


def _adapter_body(g_ref, w_ref, b_ref, o_ref):
    w = w_ref[...]
    bb = b_ref[...]
    for l in range(o_ref.shape[0]):
        x = g_ref[:, l, :]
        y = jnp.dot(x, w, preferred_element_type=jnp.float32) + bb
        o_ref[l, :, :] = y.T


def _adapter(g3, W, b, N, L, D):
    NB = 256
    assert N % NB == 0
    return pl.pallas_call(
        _adapter_body,
        grid=(N // NB,),
        in_specs=[
            pl.BlockSpec((NB, L, D), lambda i: (i, 0, 0)),
            pl.BlockSpec((D, D), lambda i: (0, 0)),
            pl.BlockSpec((1, D), lambda i: (0, 0)),
        ],
        out_specs=pl.BlockSpec((L, D, NB), lambda i: (0, 0, i)),
        out_shape=jax.ShapeDtypeStruct((L, D, N), jnp.float32),
    )(g3, W, b.reshape(1, D))


def kernel(indices, table, W, b):
    V, D = table.shape
    N, L = indices.shape
    idx = indices.reshape(-1).astype(jnp.int32)
    g = _sc_gather_flat(table, idx, N * L, D, 1600)
    g3 = g.reshape(N, L, D)
    out_t = _adapter(g3, W, b, N, L, D)
    return jnp.transpose(out_t, (2, 0, 1))
